# baseline (speedup 1.0000x reference)
import functools

import jax
import jax.numpy as jnp
from jax import lax
from jax.experimental import pallas as pl
from jax.experimental.pallas import tpu as pltpu
from jax.experimental.pallas import tpu_sc as plsc

N = 10000
E = 320000
P = 30000
IN_CH = 9
LATENT = 128
K = 512
LEVELS = 3
COMMIT = 0.25

NC = 2
NS = 16
NW = NC * NS
SEGSZ = 8192
E_PAD = 327680
NSEG = E_PAD // SEGSZ
NSEG_PAD = 64
LISTCAP = SEGSZ + 128
NPW = 320
N_ACC = NPW * NW

PPAD = 30720
PW = PPAD // NW

BLK = 1000
GRID = N // BLK

_SC_MESH = dict(core_axis_name="c", subcore_axis_name="s")


def _route_body(src_hbm, dst_hbm, lsrc, ldst, counts,
                srcb, dstb, csrc, cdst, cntb):
    cid = lax.axis_index("c")
    sid = lax.axis_index("s")
    wid = cid * NS + sid
    lo = wid * NPW
    hi = lo + NPW
    zvec = jnp.zeros((16,), jnp.int32)

    def seg_body(seg, carry):
        pltpu.sync_copy(src_hbm.at[pl.ds(seg * SEGSZ, SEGSZ)], srcb)
        pltpu.sync_copy(dst_hbm.at[pl.ds(seg * SEGSZ, SEGSZ)], dstb)

        def grp(g, cnt):
            dvec = dstb[pl.ds(g * 16, 16)]
            svec = srcb[pl.ds(g * 16, 16)]
            mask = (dvec >= lo) & (dvec < hi)
            plsc.store_compressed(csrc.at[pl.ds(cnt, 16)], svec, mask=mask)
            plsc.store_compressed(cdst.at[pl.ds(cnt, 16)], dvec, mask=mask)
            return cnt + plsc.all_reduce_population_count(mask)[0]

        cnt = lax.fori_loop(0, SEGSZ // 16, grp, jnp.int32(0))
        for i in range(8):
            csrc[pl.ds(cnt + 16 * i, 16)] = zvec
        lane0 = lax.broadcasted_iota(jnp.int32, (16,), 0) == 0
        plsc.store_compressed(cntb.at[pl.ds(seg, 16)],
                              jnp.zeros((16,), jnp.int32) + cnt, mask=lane0)
        pltpu.sync_copy(csrc.at[pl.ds(0, SEGSZ)], lsrc.at[wid, seg])
        pltpu.sync_copy(cdst.at[pl.ds(0, SEGSZ)], ldst.at[wid, seg])
        return carry

    lax.fori_loop(0, NSEG, seg_body, 0)
    pltpu.sync_copy(cntb, counts.at[wid])


_route = pl.kernel(
    _route_body,
    out_type=[
        jax.ShapeDtypeStruct((NW, NSEG, SEGSZ), jnp.int32),
        jax.ShapeDtypeStruct((NW, NSEG, SEGSZ), jnp.int32),
        jax.ShapeDtypeStruct((NW, NSEG_PAD), jnp.int32),
    ],
    mesh=plsc.VectorSubcoreMesh(**_SC_MESH),
    scratch_types=[
        pltpu.VMEM((SEGSZ,), jnp.int32),
        pltpu.VMEM((SEGSZ,), jnp.int32),
        pltpu.VMEM((LISTCAP,), jnp.int32),
        pltpu.VMEM((LISTCAP,), jnp.int32),
        pltpu.VMEM((NSEG_PAD,), jnp.int32),
    ],
    compiler_params=pltpu.CompilerParams(needs_layout_passes=False),
)


def _make_addpass(D):
    def body(table, lsrc, ldst, counts, zeros, out,
             csrc, cdst, cntb, rows, acc):
        cid = lax.axis_index("c")
        sid = lax.axis_index("s")
        wid = cid * NS + sid
        lo = wid * NPW
        pltpu.sync_copy(zeros, acc)
        pltpu.sync_copy(counts.at[wid], cntb)

        def seg_body(seg, carry):
            c = cntb[pl.ds(seg, 16)][0]

            @pl.when(c > 0)
            def _():
                @pl.when(c <= 2048)
                def _():
                    pltpu.sync_copy(lsrc.at[wid, seg, pl.ds(0, 2048)],
                                    csrc.at[pl.ds(0, 2048)])
                    pltpu.sync_copy(ldst.at[wid, seg, pl.ds(0, 2048)],
                                    cdst.at[pl.ds(0, 2048)])

                @pl.when(c > 2048)
                def _():
                    pltpu.sync_copy(lsrc.at[wid, seg], csrc.at[pl.ds(0, SEGSZ)])
                    pltpu.sync_copy(ldst.at[wid, seg], cdst.at[pl.ds(0, SEGSZ)])

                nsub = (c + 127) >> 7

                def sub(k, carry2):
                    pltpu.sync_copy(table.at[csrc.at[pl.ds(k * 128, 128)]],
                                    rows)
                    m = jnp.minimum(c - k * 128, 128)

                    def edge(e, carry3):
                        dl = cdst[pl.ds(k * 128 + e, 16)][0] - lo
                        for kk in range(D // 16):
                            sl = pl.ds(kk * 16, 16)
                            acc[dl, sl] = acc[dl, sl] + rows[e, sl]
                        return carry3

                    lax.fori_loop(0, m, edge, 0)
                    return carry2

                lax.fori_loop(0, nsub, sub, 0)

            return carry

        lax.fori_loop(0, NSEG, seg_body, 0)
        pltpu.sync_copy(acc, out.at[pl.ds(wid * NPW, NPW)])

    return pl.kernel(
        body,
        out_type=jax.ShapeDtypeStruct((N_ACC, D), jnp.float32),
        mesh=plsc.VectorSubcoreMesh(**_SC_MESH),
        scratch_types=[
            pltpu.VMEM((SEGSZ + 16,), jnp.int32),
            pltpu.VMEM((SEGSZ + 16,), jnp.int32),
            pltpu.VMEM((NSEG_PAD,), jnp.int32),
            pltpu.VMEM((128, D), jnp.float32),
            pltpu.VMEM((NPW, D), jnp.float32),
        ],
        compiler_params=pltpu.CompilerParams(use_tc_tiling_on_sc=False,
                                             needs_layout_passes=False),
    )


_addpass16 = _make_addpass(16)
_addpass128 = _make_addpass(128)


def _cons_body(recon_flat, fa, fb, out, rbuf, fabuf, fbbuf, accbuf):
    cid = lax.axis_index("c")
    sid = lax.axis_index("s")
    wid = cid * NS + sid
    pltpu.sync_copy(recon_flat, rbuf)
    pltpu.sync_copy(fa.at[pl.ds(wid * PW, PW)], fabuf)
    pltpu.sync_copy(fb.at[pl.ds(wid * PW, PW)], fbbuf)

    def step(i, acc):
        ia = fabuf[pl.ds(i * 16, 16)]
        ib = fbbuf[pl.ds(i * 16, 16)]
        for k in range(3):
            va = plsc.load_gather(rbuf, [ia + k])
            vb = plsc.load_gather(rbuf, [ib + k])
            dv = va - vb
            acc = acc + dv * dv
        return acc

    acc = lax.fori_loop(0, PW // 16, step, jnp.zeros((16,), jnp.float32))
    accbuf[...] = acc
    pltpu.sync_copy(accbuf, out.at[wid])


_cons = pl.kernel(
    _cons_body,
    out_type=jax.ShapeDtypeStruct((NW, 16), jnp.float32),
    mesh=plsc.VectorSubcoreMesh(**_SC_MESH),
    scratch_types=[
        pltpu.VMEM((N * IN_CH,), jnp.float32),
        pltpu.VMEM((PW,), jnp.int32),
        pltpu.VMEM((PW,), jnp.int32),
        pltpu.VMEM((16,), jnp.float32),
    ],
    compiler_params=pltpu.CompilerParams(needs_layout_passes=False),
)


def _layer1_body(x_ref, a_ref, w_ref, b_ref, h_ref, deg_ref):
    agg = a_ref[...]
    deg = jnp.clip(agg[:, 9:10], 1.0, None)
    xin = x_ref[...] + agg[:, 0:9] / deg
    h = jnp.dot(xin, w_ref[...], preferred_element_type=jnp.float32) + b_ref[...]
    h_ref[...] = jnp.maximum(h, 0.0)
    deg_ref[...] = deg


def _tc_layer1(x, agg, W1, b1):
    return pl.pallas_call(
        _layer1_body,
        grid=(GRID,),
        in_specs=[
            pl.BlockSpec((BLK, IN_CH), lambda i: (i, 0)),
            pl.BlockSpec((BLK, 16), lambda i: (i, 0)),
            pl.BlockSpec((IN_CH, LATENT), lambda i: (0, 0)),
            pl.BlockSpec((1, LATENT), lambda i: (0, 0)),
        ],
        out_specs=[
            pl.BlockSpec((BLK, LATENT), lambda i: (i, 0)),
            pl.BlockSpec((BLK, 1), lambda i: (i, 0)),
        ],
        out_shape=[
            jax.ShapeDtypeStruct((N, LATENT), jnp.float32),
            jax.ShapeDtypeStruct((N, 1), jnp.float32),
        ],
    )(x, agg, W1, b1.reshape(1, LATENT))


def _layer2_body(h_ref, a_ref, deg_ref, w_ref, b_ref, ze_ref):
    ze_ref[...] = (jnp.dot(h_ref[...] + a_ref[...] / deg_ref[...], w_ref[...],
                           preferred_element_type=jnp.float32) + b_ref[...])


def _tc_layer2(h, agg, deg, W2, b2):
    return pl.pallas_call(
        _layer2_body,
        grid=(GRID,),
        in_specs=[
            pl.BlockSpec((BLK, LATENT), lambda i: (i, 0)),
            pl.BlockSpec((BLK, LATENT), lambda i: (i, 0)),
            pl.BlockSpec((BLK, 1), lambda i: (i, 0)),
            pl.BlockSpec((LATENT, LATENT), lambda i: (0, 0)),
            pl.BlockSpec((1, LATENT), lambda i: (0, 0)),
        ],
        out_specs=pl.BlockSpec((BLK, LATENT), lambda i: (i, 0)),
        out_shape=jax.ShapeDtypeStruct((N, LATENT), jnp.float32),
    )(h, agg, deg, W2, b2.reshape(1, LATENT))


def _vq_body(z_ref, cb_ref, zq_ref, idx_ref, loss_ref):
    pid = pl.program_id(0)
    z = z_ref[...]
    iota = lax.broadcasted_iota(jnp.int32, (BLK, K), 1)
    r = z
    quant = jnp.zeros_like(z)
    lsum = jnp.float32(0.0)
    idxs = []
    for l in range(LEVELS):
        cbl = cb_ref[l]
        cbh = cbl.astype(jnp.bfloat16).astype(jnp.float32)
        cbm = (cbl - cbh).astype(jnp.bfloat16).astype(jnp.float32)
        cblo = cbl - cbh - cbm
        cb2 = jnp.sum(cbl * cbl, axis=1)
        r2 = jnp.sum(r * r, axis=1, keepdims=True)
        d = (r2 - 2.0 * lax.dot_general(r, cbl, (((1,), (1,)), ((), ())),
                                        preferred_element_type=jnp.float32)
             + cb2[None, :])
        m = jnp.min(d, axis=1, keepdims=True)
        idx = jnp.min(jnp.where(d == m, iota, K), axis=1)
        oh = (iota == idx[:, None]).astype(jnp.float32)
        q = (jnp.dot(oh, cbh, preferred_element_type=jnp.float32)
             + jnp.dot(oh, cbm, preferred_element_type=jnp.float32)
             + jnp.dot(oh, cblo, preferred_element_type=jnp.float32))
        diff = r - q
        lsum = lsum + jnp.sum(diff * diff)
        quant = quant + q
        r = diff
        idxs.append(idx)
    zq_ref[...] = z + (quant - z)
    idx_ref[...] = jnp.stack(idxs, axis=1)
    prev = jnp.where(pid == 0, jnp.zeros((1, 1), jnp.float32), loss_ref[...])
    tot = prev + lsum
    loss_ref[...] = jnp.where(pid == GRID - 1,
                              tot * ((1.0 + COMMIT) / (N * LATENT)), tot)


def _tc_vq(z_e, codebooks):
    return pl.pallas_call(
        _vq_body,
        grid=(GRID,),
        in_specs=[
            pl.BlockSpec((BLK, LATENT), lambda i: (i, 0)),
            pl.BlockSpec((LEVELS, K, LATENT), lambda i: (0, 0, 0)),
        ],
        out_specs=[
            pl.BlockSpec((BLK, LATENT), lambda i: (i, 0)),
            pl.BlockSpec((BLK, LEVELS), lambda i: (i, 0)),
            pl.BlockSpec((1, 1), lambda i: (0, 0)),
        ],
        out_shape=[
            jax.ShapeDtypeStruct((N, LATENT), jnp.float32),
            jax.ShapeDtypeStruct((N, LEVELS), jnp.int32),
            jax.ShapeDtypeStruct((1, 1), jnp.float32),
        ],
    )(z_e, codebooks)


def _layer3_body(zq_ref, a_ref, deg_ref, w3_ref, b3_ref, w4_ref, r4_ref):
    h2 = (jnp.dot(zq_ref[...] + a_ref[...] / deg_ref[...], w3_ref[...],
                  preferred_element_type=jnp.float32) + b3_ref[...])
    h2 = jnp.maximum(h2, 0.0)
    r4_ref[...] = jnp.dot(h2, w4_ref[...], preferred_element_type=jnp.float32)


def _tc_layer3(z_q, agg, deg, W3, b3, W4pad):
    return pl.pallas_call(
        _layer3_body,
        grid=(GRID,),
        in_specs=[
            pl.BlockSpec((BLK, LATENT), lambda i: (i, 0)),
            pl.BlockSpec((BLK, LATENT), lambda i: (i, 0)),
            pl.BlockSpec((BLK, 1), lambda i: (i, 0)),
            pl.BlockSpec((LATENT, LATENT), lambda i: (0, 0)),
            pl.BlockSpec((1, LATENT), lambda i: (0, 0)),
            pl.BlockSpec((LATENT, 16), lambda i: (0, 0)),
        ],
        out_specs=pl.BlockSpec((BLK, 16), lambda i: (i, 0)),
        out_shape=jax.ShapeDtypeStruct((N, 16), jnp.float32),
    )(z_q, agg, deg, W3, b3.reshape(1, LATENT), W4pad)


def _final_body(r4_ref, a_ref, deg_ref, b4_ref, y0_ref, y1_ref, y2_ref,
                recon_ref, rl_ref):
    pid = pl.program_id(0)
    rec = r4_ref[...] + a_ref[...] / deg_ref[...] + b4_ref[...]
    recon_ref[...] = rec[:, 0:9]
    s0 = jnp.sum(jnp.abs(rec - y0_ref[...]), axis=1, keepdims=True)
    s1 = jnp.sum(jnp.abs(rec - y1_ref[...]), axis=1, keepdims=True)
    s2 = jnp.sum(jnp.abs(rec - y2_ref[...]), axis=1, keepdims=True)
    mn = jnp.minimum(jnp.minimum(s0, s1), s2)
    part = jnp.sum(mn)
    prev = jnp.where(pid == 0, jnp.zeros((1, 1), jnp.float32), rl_ref[...])
    tot = prev + part
    rl_ref[...] = jnp.where(pid == GRID - 1, tot / (9.0 * N), tot)


def _tc_final(r4, agg, deg, b4pad, y0, y1, y2):
    return pl.pallas_call(
        _final_body,
        grid=(GRID,),
        in_specs=[
            pl.BlockSpec((BLK, 16), lambda i: (i, 0)),
            pl.BlockSpec((BLK, 16), lambda i: (i, 0)),
            pl.BlockSpec((BLK, 1), lambda i: (i, 0)),
            pl.BlockSpec((1, 16), lambda i: (0, 0)),
            pl.BlockSpec((BLK, 16), lambda i: (i, 0)),
            pl.BlockSpec((BLK, 16), lambda i: (i, 0)),
            pl.BlockSpec((BLK, 16), lambda i: (i, 0)),
        ],
        out_specs=[
            pl.BlockSpec((BLK, IN_CH), lambda i: (i, 0)),
            pl.BlockSpec((1, 1), lambda i: (0, 0)),
        ],
        out_shape=[
            jax.ShapeDtypeStruct((N, IN_CH), jnp.float32),
            jax.ShapeDtypeStruct((1, 1), jnp.float32),
        ],
    )(r4, agg, deg, b4pad, y0, y1, y2)


def _scalars_body(rl_ref, vq_ref, cp_ref, cons_ref, tot_ref):
    c = jnp.sum(cp_ref[...], keepdims=True).reshape(1, 1) / (3.0 * P)
    cons_ref[...] = c
    tot_ref[...] = rl_ref[...] + vq_ref[...] + 0.3 * c


def _tc_scalars(rl, vq, cons_parts):
    return pl.pallas_call(
        _scalars_body,
        out_shape=[
            jax.ShapeDtypeStruct((1, 1), jnp.float32),
            jax.ShapeDtypeStruct((1, 1), jnp.float32),
        ],
    )(rl, vq, cons_parts)


def kernel(x, edge_index, y, sv_tri_a, sv_local_a, sv_tri_b, sv_local_b,
           W1, b1, W2, b2, W3, b3, W4, b4, codebooks):
    pad = E_PAD - E
    src_flat = jnp.concatenate([edge_index[0], jnp.zeros((pad,), jnp.int32)])
    dst_flat = jnp.concatenate(
        [edge_index[1], jnp.full((pad,), 2 ** 30, jnp.int32)])

    lsrc, ldst, counts = _route(src_flat, dst_flat)

    xpad = jnp.concatenate(
        [x, jnp.ones((N, 1), jnp.float32), jnp.zeros((N, 6), jnp.float32)],
        axis=1)
    zeros16 = jnp.zeros((NPW, 16), jnp.float32)
    zeros128 = jnp.zeros((NPW, 128), jnp.float32)

    agg1 = _addpass16(xpad, lsrc, ldst, counts, zeros16)
    h, deg = _tc_layer1(x, agg1, W1, b1)

    agg2 = _addpass128(h, lsrc, ldst, counts, zeros128)
    z_e = _tc_layer2(h, agg2, deg, W2, b2)
    z_q, indices, vq11 = _tc_vq(z_e, codebooks)

    agg3 = _addpass128(z_q, lsrc, ldst, counts, zeros128)
    W4pad = jnp.pad(W4, ((0, 0), (0, 16 - IN_CH)))
    r4 = _tc_layer3(z_q, agg3, deg, W3, b3, W4pad)

    agg4 = _addpass16(r4, lsrc, ldst, counts, zeros16)
    b4pad = jnp.pad(b4, (0, 16 - IN_CH)).reshape(1, 16)
    y3 = y.reshape(-1, 3, 3)
    y0 = jnp.pad(y, ((0, 0), (0, 16 - IN_CH)))
    y1 = jnp.pad(y3[:, jnp.array([1, 2, 0]), :].reshape(N, IN_CH),
                 ((0, 0), (0, 16 - IN_CH)))
    y2 = jnp.pad(y3[:, jnp.array([2, 0, 1]), :].reshape(N, IN_CH),
                 ((0, 0), (0, 16 - IN_CH)))
    recon, rl11 = _tc_final(r4, agg4, deg, b4pad, y0, y1, y2)

    fa = sv_tri_a * IN_CH + sv_local_a * 3
    fb = sv_tri_b * IN_CH + sv_local_b * 3
    fa = jnp.pad(fa, (0, PPAD - P))
    fb = jnp.pad(fb, (0, PPAD - P))
    cons_parts = _cons(recon.reshape(-1), fa, fb)

    cons11, tot11 = _tc_scalars(rl11, vq11, cons_parts)

    return (recon, rl11.reshape(()), vq11.reshape(()), cons11.reshape(()),
            tot11.reshape(()), indices, z_e, z_q)
